# Initial kernel scaffold; baseline (speedup 1.0000x reference)
import jax
import jax.numpy as jnp
from jax.experimental import pallas as pl


def kernel(input, selector, weight, bias):
    raise NotImplementedError("write your pallas kernel here")


# speedup vs baseline: 6.9040x; 6.9040x over previous
import jax
import jax.numpy as jnp
from jax.experimental import pallas as pl
from jax.experimental.pallas import tpu as pltpu

IN_C = 96
OUT_C = 96
BREADTH = 8
N_BLK = 1792


def _fused_kernel(x_ref, sel_ref, w_ref, b_ref, o_ref):
    x = x_ref[0]
    s = sel_ref[0]
    m = jnp.max(s, axis=0, keepdims=True)
    e = jnp.exp(s - m)
    gate = e / jnp.sum(e, axis=0, keepdims=True)

    w = w_ref[...]
    y = jax.lax.dot_general(
        w, x, (((1,), (0,)), ((), ())),
        preferred_element_type=jnp.float32)

    acc = jnp.broadcast_to(b_ref[...], (OUT_C, x.shape[1]))
    for si in range(BREADTH):
        acc = acc + gate[si:si + 1, :] * y[si * OUT_C:(si + 1) * OUT_C, :]
    o_ref[0] = acc


def kernel(input, selector, weight, bias):
    b, c, h, w_dim = input.shape
    hw = h * w_dim
    x = input.reshape(b, c, hw)
    sel = selector.reshape(b, BREADTH, hw)
    w = jnp.transpose(weight[:, :, :, 0, 0], (2, 0, 1)).reshape(BREADTH * OUT_C, IN_C)
    b2 = bias.reshape(OUT_C, 1)

    grid = (b, hw // N_BLK)
    out = pl.pallas_call(
        _fused_kernel,
        grid=grid,
        in_specs=[
            pl.BlockSpec((1, IN_C, N_BLK), lambda i, j: (i, 0, j)),
            pl.BlockSpec((1, BREADTH, N_BLK), lambda i, j: (i, 0, j)),
            pl.BlockSpec((BREADTH * OUT_C, IN_C), lambda i, j: (0, 0)),
            pl.BlockSpec((OUT_C, 1), lambda i, j: (0, 0)),
        ],
        out_specs=pl.BlockSpec((1, OUT_C, N_BLK), lambda i, j: (i, 0, j)),
        out_shape=jax.ShapeDtypeStruct((b, OUT_C, hw), jnp.float32),
        compiler_params=pltpu.CompilerParams(
            dimension_semantics=("parallel", "parallel")),
    )(x, sel, w, b2)
    return out.reshape(b, OUT_C, h, w_dim)
